# baseline (speedup 1.0000x reference)
import jax
import jax.numpy as jnp
from jax.experimental import pallas as pl

_N = 1024
_F32 = jnp.float32


def _fused(adj_ref, x_ref, W1_ref, b1_ref, W2_ref, b2_ref, Wres_ref,
           bres_ref, Wfc1_ref, bfc1_ref, Wfc2_ref, bfc2_ref,
           x_out_ref, A_out_ref):
    adj = adj_ref[...]
    x0 = x_ref[...]

    adjb = adj.astype(jnp.bfloat16)

    deg = jnp.transpose(jnp.sum(adj, axis=0, keepdims=True)) + 2.0
    dinv = jax.lax.rsqrt(deg)
    dinv2 = 2.0 * dinv * dinv

    def conv(x, W_ref, b_ref):
        xw = jnp.dot(x, W_ref[...], preferred_element_type=_F32)
        y = dinv * xw
        y_hi = y.astype(jnp.bfloat16)
        y_lo = (y - y_hi.astype(_F32)).astype(jnp.bfloat16)
        dn = (((0,), (0,)), ((), ()))
        t = (jax.lax.dot_general(adjb, y_hi, dn, preferred_element_type=_F32)
             + jax.lax.dot_general(adjb, y_lo, dn,
                                   preferred_element_type=_F32))
        return dinv * t + dinv2 * xw + b_ref[...]

    def bdot(a, W_ref):
        return jnp.dot(a.astype(jnp.bfloat16), W_ref[...].astype(jnp.bfloat16),
                       preferred_element_type=_F32)

    x1 = jax.nn.relu(conv(x0, W1_ref, b1_ref))
    x2 = jax.nn.relu(conv(x1, W2_ref, b2_ref))
    x = x2 + jnp.dot(x1, Wres_ref[...], preferred_element_type=_F32) \
        + bres_ref[...]
    h = jax.nn.relu(jnp.dot(x, Wfc1_ref[...], preferred_element_type=_F32)
                    + bfc1_ref[...])
    A = jnp.dot(h, Wfc2_ref[...], preferred_element_type=_F32) + bfc2_ref[...]

    x_out_ref[...] = x
    A_out_ref[...] = A


def kernel(adj, node_emb, W1, b1, W2, b2, Wres, bres, Wfc1, bfc1, Wfc2, bfc2):
    out = pl.pallas_call(
        _fused,
        out_shape=(
            jax.ShapeDtypeStruct((_N, 128), _F32),
            jax.ShapeDtypeStruct((_N, 1), _F32),
        ),
    )(adj, node_emb,
      W1, b1.reshape(1, -1), W2, b2.reshape(1, -1),
      Wres, bres.reshape(1, -1), Wfc1, bfc1.reshape(1, -1),
      Wfc2, bfc2.reshape(1, -1))
    return out
